# baseline (speedup 1.0000x reference)
import functools

import jax
import jax.numpy as jnp
from jax import lax
from jax.experimental import pallas as pl
from jax.experimental.pallas import tpu as pltpu
from jax.experimental.pallas import tpu_sc as plsc

N_NODES = 10000
N_EDGES = 320000
NODE_DIM = 128
EDGE_DIM = 16
HIDDEN = 128
N_PAD = N_NODES + 16

NC = 2
NS = 16
NW = NC * NS
CHUNK = 96
K = 106
E_PAD = NW * K * CHUNK
KH = K // 2
E_HALF = NW * KH * CHUNK

F32 = jnp.float32


def _prep_body(Wm1, We, be, bm1, Wm2, Wu1, bm2,
               We2_o, bias1_o, Wcombo_o, bcombo_o):
    Wm1_bot = Wm1[HIDDEN:, :]
    Wu1_bot = Wu1[NODE_DIM:, :]
    We2_o[...] = jnp.dot(We[...], Wm1_bot, preferred_element_type=F32)
    bias1_o[...] = bm1[...] + jnp.dot(be[...], Wm1_bot,
                                      preferred_element_type=F32)
    Wcombo_o[...] = jnp.dot(Wm2[...], Wu1_bot, preferred_element_type=F32)
    bcombo_o[...] = jnp.dot(bm2[...], Wu1_bot, preferred_element_type=F32)


def _hm_body(nf, Wn, bn, Wm1, bias1, hm_o):
    h = jnp.dot(nf[...], Wn[...], preferred_element_type=F32) + bn[...]
    hm_o[...] = jnp.dot(h, Wm1[:HIDDEN, :],
                        preferred_element_type=F32) + bias1[...]


def _edge_body(g, ef, We2, xt_o):
    xt_o[...] = jnp.maximum(
        g[...] + jnp.dot(ef[...], We2[...], preferred_element_type=F32), 0.0)


def _degterm_body(degs, bcombo, dt_o):
    bb = jnp.broadcast_to(bcombo[...], (2 * NW, HIDDEN))
    dt_o[...] = lax.dot_general(degs[...], bb, (((0,), (0,)), ((), ())),
                                preferred_element_type=F32)


def _post_body(nf, agg2a, agg2b, degterm, Wu1, bu1, Wcombo, Wu2, bu2, out_o):
    aggs = agg2a[0] + agg2a[1] + agg2b[0] + agg2b[1]
    pre = (jnp.dot(nf[...], Wu1[:NODE_DIM, :], preferred_element_type=F32)
           + jnp.dot(aggs, Wcombo[...], preferred_element_type=F32)
           + degterm[...] + bu1[...])
    out_o[...] = jnp.dot(jnp.maximum(pre, 0.0), Wu2[...],
                         preferred_element_type=F32) + bu2[...]


def _gather_body(hm_hbm, idx_hbm, g_hbm, idx_v, buf0, buf1,
                 gsem0, gsem1, wsem0, wsem1, hm_sh):
    sid = lax.axis_index("s")
    wid = sid * NC + lax.axis_index("c")

    @pl.when(sid == 0)
    def _():
        pltpu.sync_copy(hm_hbm, hm_sh)

    plsc.subcore_barrier()
    pltpu.sync_copy(idx_hbm.at[wid], idx_v)
    base = wid * (KH * CHUNK)
    bufs = (buf0, buf1)
    gsems = (gsem0, gsem1)
    wsems = (wsem0, wsem1)

    def out_slice(j):
        return g_hbm.at[pl.ds(base + j * CHUNK, CHUNK)]

    pltpu.async_copy(hm_sh.at[idx_v.at[0]], buf0, gsem0)

    def body(jj, carry):
        for t in range(2):
            j = jj * 2 + t
            b = t

            @pl.when(j < KH)
            def _():
                pltpu.make_async_copy(
                    hm_sh.at[idx_v.at[j]], bufs[b], gsems[b]).wait()

                @pl.when(j + 1 < KH)
                def _():
                    @pl.when(j >= 1)
                    def _():
                        pltpu.make_async_copy(
                            bufs[1 - b], out_slice(j - 1),
                            wsems[1 - b]).wait()

                    pltpu.async_copy(
                        hm_sh.at[idx_v.at[j + 1]], bufs[1 - b], gsems[1 - b])

                pltpu.async_copy(bufs[b], out_slice(j), wsems[b])

        return carry

    lax.fori_loop(0, (KH + 1) // 2, body, 0)
    pltpu.make_async_copy(bufs[(KH - 2) % 2], out_slice(KH - 2),
                          wsems[(KH - 2) % 2]).wait()
    pltpu.make_async_copy(bufs[(KH - 1) % 2], out_slice(KH - 1),
                          wsems[(KH - 1) % 2]).wait()


def _scatter_body(xt_hbm, idx_hbm, zero_hbm, agg_hbm, degs_hbm,
                  idx_v, buf0, buf1, deg_v, sem0, sem1, acc_sh):
    cid = lax.axis_index("c")
    sid = lax.axis_index("s")
    wid = sid * NC + cid

    @pl.when(sid == 0)
    def _():
        pltpu.sync_copy(zero_hbm, acc_sh)

    zvec = jnp.zeros((16,), F32)

    def zbody(i, carry):
        deg_v[pl.ds(i * 16, 16)] = zvec
        return carry

    lax.fori_loop(0, N_PAD // 16, zbody, 0)

    plsc.subcore_barrier()
    pltpu.sync_copy(idx_hbm.at[wid], idx_v)
    base = wid * (KH * CHUNK)
    bufs = (buf0, buf1)
    sems = (sem0, sem1)
    ones = jnp.ones((16,), F32)

    pltpu.async_copy(xt_hbm.at[pl.ds(base, CHUNK)], buf0, sem0)

    def body(jj, carry):
        for t in range(2):
            j = jj * 2 + t
            b = t

            @pl.when(j < KH)
            def _():
                pltpu.make_async_copy(
                    xt_hbm.at[pl.ds(base + j * CHUNK, CHUNK)],
                    bufs[b], sems[b]).wait()

                @pl.when(j + 1 < KH)
                def _():
                    pltpu.async_copy(
                        xt_hbm.at[pl.ds(base + (j + 1) * CHUNK, CHUNK)],
                        bufs[1 - b], sems[1 - b])

                pltpu.sync_copy(bufs[b], acc_sh.at[idx_v.at[j]], add=True)

                for q in range(CHUNK // 16):
                    dvec = idx_v[j, pl.ds(q * 16, 16)]
                    plsc.addupdate_scatter(deg_v, [dvec], ones)

        return carry

    lax.fori_loop(0, (KH + 1) // 2, body, 0)
    plsc.subcore_barrier()

    @pl.when(sid == 0)
    def _():
        pltpu.sync_copy(acc_sh, agg_hbm.at[cid])

    pltpu.sync_copy(deg_v, degs_hbm.at[wid])


def kernel(node_feats, edge_idx, edge_feats, Wn, bn, We, be,
           Wm1, bm1, Wm2, bm2, Wu1, bu1, Wu2, bu2):
    bn2 = bn.reshape(1, HIDDEN)
    be2 = be.reshape(1, HIDDEN)
    bm12 = bm1.reshape(1, HIDDEN)
    bm22 = bm2.reshape(1, HIDDEN)
    bu12 = bu1.reshape(1, HIDDEN)
    bu22 = bu2.reshape(1, HIDDEN)

    We2, bias1, Wcombo, bcombo = pl.pallas_call(
        _prep_body,
        out_shape=[
            jax.ShapeDtypeStruct((EDGE_DIM, HIDDEN), F32),
            jax.ShapeDtypeStruct((1, HIDDEN), F32),
            jax.ShapeDtypeStruct((HIDDEN, HIDDEN), F32),
            jax.ShapeDtypeStruct((1, HIDDEN), F32),
        ],
    )(Wm1, We, be2, bm12, Wm2, Wu1, bm22)

    BLK_N = 1000
    hm = pl.pallas_call(
        _hm_body,
        grid=(N_NODES // BLK_N,),
        in_specs=[
            pl.BlockSpec((BLK_N, NODE_DIM), lambda i: (i, 0)),
            pl.BlockSpec((NODE_DIM, HIDDEN), lambda i: (0, 0)),
            pl.BlockSpec((1, HIDDEN), lambda i: (0, 0)),
            pl.BlockSpec((2 * HIDDEN, HIDDEN), lambda i: (0, 0)),
            pl.BlockSpec((1, HIDDEN), lambda i: (0, 0)),
        ],
        out_specs=pl.BlockSpec((BLK_N, HIDDEN), lambda i: (i, 0)),
        out_shape=jax.ShapeDtypeStruct((N_NODES, HIDDEN), F32),
    )(node_feats, Wn, bn2, Wm1, bias1)

    pad = E_PAD - N_EDGES
    src = jnp.concatenate(
        [edge_idx[0].astype(jnp.int32), jnp.zeros((pad,), jnp.int32)])
    dst = jnp.concatenate(
        [edge_idx[1].astype(jnp.int32), jnp.full((pad,), N_NODES, jnp.int32)])
    src3 = src.reshape(NW, K, CHUNK)
    dst3 = dst.reshape(NW, K, CHUNK)
    ef_p = jnp.concatenate(
        [edge_feats, jnp.zeros((pad, EDGE_DIM), F32)], axis=0)

    mesh = plsc.VectorSubcoreMesh(core_axis_name="c", subcore_axis_name="s")

    gather_call = pl.kernel(
        _gather_body,
        out_type=jax.ShapeDtypeStruct((E_HALF, HIDDEN), F32),
        mesh=mesh,
        scratch_types=[
            pltpu.VMEM((KH, CHUNK), jnp.int32),
            pltpu.VMEM((CHUNK, HIDDEN), F32),
            pltpu.VMEM((CHUNK, HIDDEN), F32),
            pltpu.SemaphoreType.DMA,
            pltpu.SemaphoreType.DMA,
            pltpu.SemaphoreType.DMA,
            pltpu.SemaphoreType.DMA,
            pltpu.VMEM_SHARED((N_NODES, HIDDEN), F32),
        ],
    )

    BLK_E = 3072
    edge_call = pl.pallas_call(
        _edge_body,
        grid=(E_HALF // BLK_E,),
        in_specs=[
            pl.BlockSpec((BLK_E, HIDDEN), lambda i: (i, 0)),
            pl.BlockSpec((BLK_E, EDGE_DIM), lambda i: (i, 0)),
            pl.BlockSpec((EDGE_DIM, HIDDEN), lambda i: (0, 0)),
        ],
        out_specs=pl.BlockSpec((BLK_E, HIDDEN), lambda i: (i, 0)),
        out_shape=jax.ShapeDtypeStruct((E_HALF, HIDDEN), F32),
    )

    zeros128 = jnp.zeros((N_PAD, HIDDEN), F32)
    scatter_call = pl.kernel(
        _scatter_body,
        out_type=[
            jax.ShapeDtypeStruct((NC, N_PAD, HIDDEN), F32),
            jax.ShapeDtypeStruct((NW, N_PAD), F32),
        ],
        mesh=mesh,
        scratch_types=[
            pltpu.VMEM((KH, CHUNK), jnp.int32),
            pltpu.VMEM((CHUNK, HIDDEN), F32),
            pltpu.VMEM((CHUNK, HIDDEN), F32),
            pltpu.VMEM((N_PAD,), F32),
            pltpu.SemaphoreType.DMA,
            pltpu.SemaphoreType.DMA,
            pltpu.VMEM_SHARED((N_PAD, HIDDEN), F32),
        ],
        compiler_params=pltpu.CompilerParams(needs_layout_passes=False),
    )

    ef4 = ef_p.reshape(NW, K, CHUNK, EDGE_DIM)
    agg_halves = []
    degs_halves = []
    for h in range(2):
        src3h = src3[:, h * KH:(h + 1) * KH]
        dst3h = dst3[:, h * KH:(h + 1) * KH]
        efh = ef4[:, h * KH:(h + 1) * KH].reshape(E_HALF, EDGE_DIM)
        gh = gather_call(hm, src3h)
        xth = edge_call(gh, efh, We2)
        agg2h, degsh = scatter_call(xth, dst3h, zeros128)
        agg_halves.append(agg2h)
        degs_halves.append(degsh)

    degs_all = jnp.concatenate(degs_halves, axis=0)
    degterm = pl.pallas_call(
        _degterm_body,
        out_shape=jax.ShapeDtypeStruct((N_PAD, HIDDEN), F32),
    )(degs_all, bcombo)

    out = pl.pallas_call(
        _post_body,
        grid=(N_NODES // BLK_N,),
        in_specs=[
            pl.BlockSpec((BLK_N, NODE_DIM), lambda i: (i, 0)),
            pl.BlockSpec((NC, BLK_N, HIDDEN), lambda i: (0, i, 0)),
            pl.BlockSpec((NC, BLK_N, HIDDEN), lambda i: (0, i, 0)),
            pl.BlockSpec((BLK_N, HIDDEN), lambda i: (i, 0)),
            pl.BlockSpec((NODE_DIM + HIDDEN, HIDDEN), lambda i: (0, 0)),
            pl.BlockSpec((1, HIDDEN), lambda i: (0, 0)),
            pl.BlockSpec((HIDDEN, HIDDEN), lambda i: (0, 0)),
            pl.BlockSpec((HIDDEN, HIDDEN), lambda i: (0, 0)),
            pl.BlockSpec((1, HIDDEN), lambda i: (0, 0)),
        ],
        out_specs=pl.BlockSpec((BLK_N, HIDDEN), lambda i: (i, 0)),
        out_shape=jax.ShapeDtypeStruct((N_NODES, HIDDEN), F32),
    )(node_feats, agg_halves[0], agg_halves[1], degterm,
      Wu1, bu12, Wcombo, Wu2, bu22)

    return out


# speedup vs baseline: 1.0299x; 1.0299x over previous
import functools

import jax
import jax.numpy as jnp
from jax import lax
from jax.experimental import pallas as pl
from jax.experimental.pallas import tpu as pltpu
from jax.experimental.pallas import tpu_sc as plsc

N_NODES = 10000
N_EDGES = 320000
NODE_DIM = 128
EDGE_DIM = 16
HIDDEN = 128
N_PAD = N_NODES + 16

NC = 2
NS = 16
NW = NC * NS
CHUNK = 88
K = -(-N_EDGES // (NW * CHUNK))
E_PAD = NW * K * CHUNK

F32 = jnp.float32


def _prep_body(Wm1, We, be, bm1, Wm2, Wu1, bm2,
               We2_o, bias1_o, Wcombo_o, bcombo_o):
    Wm1_bot = Wm1[HIDDEN:, :]
    Wu1_bot = Wu1[NODE_DIM:, :]
    We2_o[...] = jnp.dot(We[...], Wm1_bot, preferred_element_type=F32)
    bias1_o[...] = bm1[...] + jnp.dot(be[...], Wm1_bot,
                                      preferred_element_type=F32)
    Wcombo_o[...] = jnp.dot(Wm2[...], Wu1_bot, preferred_element_type=F32)
    bcombo_o[...] = jnp.dot(bm2[...], Wu1_bot, preferred_element_type=F32)


def _hm_body(nf, Wn, bn, Wm1, bias1, hm_o):
    h = jnp.dot(nf[...], Wn[...], preferred_element_type=F32) + bn[...]
    hm_o[...] = jnp.dot(h, Wm1[:HIDDEN, :],
                        preferred_element_type=F32) + bias1[...]


def _edge_body(g, ef, We2, xt_o):
    xt_o[...] = jnp.maximum(
        g[...] + jnp.dot(ef[...], We2[...], preferred_element_type=F32), 0.0)


def _degterm_body(degs, bcombo, dt_o):
    bb = jnp.broadcast_to(bcombo[...], (NW, HIDDEN))
    dt_o[...] = lax.dot_general(degs[...], bb, (((0,), (0,)), ((), ())),
                                preferred_element_type=F32)


def _post_body(nf, agg2, degterm, Wu1, bu1, Wcombo, Wu2, bu2, out_o):
    aggs = agg2[0] + agg2[1]
    pre = (jnp.dot(nf[...], Wu1[:NODE_DIM, :], preferred_element_type=F32)
           + jnp.dot(aggs, Wcombo[...], preferred_element_type=F32)
           + degterm[...] + bu1[...])
    out_o[...] = jnp.dot(jnp.maximum(pre, 0.0), Wu2[...],
                         preferred_element_type=F32) + bu2[...]


def _gather_body(hm_hbm, idx_hbm, g_hbm, idx_v, buf0, buf1, buf2,
                 gsem0, gsem1, gsem2, wsem0, wsem1, wsem2, hm_sh):
    sid = lax.axis_index("s")
    wid = sid * NC + lax.axis_index("c")

    @pl.when(sid == 0)
    def _():
        pltpu.sync_copy(hm_hbm, hm_sh)

    plsc.subcore_barrier()
    pltpu.sync_copy(idx_hbm.at[wid], idx_v)
    base = wid * (K * CHUNK)
    bufs = (buf0, buf1, buf2)
    gsems = (gsem0, gsem1, gsem2)
    wsems = (wsem0, wsem1, wsem2)

    def out_slice(j):
        return g_hbm.at[pl.ds(base + j * CHUNK, CHUNK)]

    pltpu.async_copy(hm_sh.at[idx_v.at[0]], buf0, gsem0)

    def body(jj, carry):
        for t in range(3):
            j = jj * 3 + t
            b = t

            @pl.when(j < K)
            def _():
                pltpu.make_async_copy(
                    hm_sh.at[idx_v.at[j]], bufs[b], gsems[b]).wait()

                nb = (b + 1) % 3

                @pl.when(j + 1 < K)
                def _():
                    @pl.when(j >= 2)
                    def _():
                        pltpu.make_async_copy(
                            bufs[nb], out_slice(j - 2), wsems[nb]).wait()

                    pltpu.async_copy(
                        hm_sh.at[idx_v.at[j + 1]], bufs[nb], gsems[nb])

                pltpu.async_copy(bufs[b], out_slice(j), wsems[b])

        return carry

    lax.fori_loop(0, (K + 2) // 3, body, 0)
    for dd in (3, 2, 1):
        j = K - dd
        pltpu.make_async_copy(bufs[j % 3], out_slice(j), wsems[j % 3]).wait()


def _scatter_body(xt_hbm, idx_hbm, zero_hbm, agg_hbm, degs_hbm,
                  idx_v, buf0, buf1, deg_v, sem0, sem1, acc_sh):
    cid = lax.axis_index("c")
    sid = lax.axis_index("s")
    wid = sid * NC + cid

    @pl.when(sid == 0)
    def _():
        pltpu.sync_copy(zero_hbm, acc_sh)

    zvec = jnp.zeros((16,), F32)

    def zbody(i, carry):
        deg_v[pl.ds(i * 16, 16)] = zvec
        return carry

    lax.fori_loop(0, N_PAD // 16, zbody, 0)

    plsc.subcore_barrier()
    pltpu.sync_copy(idx_hbm.at[wid], idx_v)
    base = wid * (K * CHUNK)
    bufs = (buf0, buf1)
    sems = (sem0, sem1)
    ones = jnp.ones((16,), F32)

    pltpu.async_copy(xt_hbm.at[pl.ds(base, CHUNK)], buf0, sem0)

    def body(jj, carry):
        for t in range(2):
            j = jj * 2 + t
            b = t

            @pl.when(j < K)
            def _():
                pltpu.make_async_copy(
                    xt_hbm.at[pl.ds(base + j * CHUNK, CHUNK)],
                    bufs[b], sems[b]).wait()

                @pl.when(j + 1 < K)
                def _():
                    pltpu.async_copy(
                        xt_hbm.at[pl.ds(base + (j + 1) * CHUNK, CHUNK)],
                        bufs[1 - b], sems[1 - b])

                pltpu.sync_copy(bufs[b], acc_sh.at[idx_v.at[j]], add=True)

                for q in range(CHUNK // 16):
                    dvec = idx_v[j, pl.ds(q * 16, 16)]
                    plsc.addupdate_scatter(deg_v, [dvec], ones)

        return carry

    lax.fori_loop(0, (K + 1) // 2, body, 0)
    plsc.subcore_barrier()

    @pl.when(sid == 0)
    def _():
        pltpu.sync_copy(acc_sh, agg_hbm.at[cid])

    pltpu.sync_copy(deg_v, degs_hbm.at[wid])


def kernel(node_feats, edge_idx, edge_feats, Wn, bn, We, be,
           Wm1, bm1, Wm2, bm2, Wu1, bu1, Wu2, bu2):
    bn2 = bn.reshape(1, HIDDEN)
    be2 = be.reshape(1, HIDDEN)
    bm12 = bm1.reshape(1, HIDDEN)
    bm22 = bm2.reshape(1, HIDDEN)
    bu12 = bu1.reshape(1, HIDDEN)
    bu22 = bu2.reshape(1, HIDDEN)

    We2, bias1, Wcombo, bcombo = pl.pallas_call(
        _prep_body,
        out_shape=[
            jax.ShapeDtypeStruct((EDGE_DIM, HIDDEN), F32),
            jax.ShapeDtypeStruct((1, HIDDEN), F32),
            jax.ShapeDtypeStruct((HIDDEN, HIDDEN), F32),
            jax.ShapeDtypeStruct((1, HIDDEN), F32),
        ],
    )(Wm1, We, be2, bm12, Wm2, Wu1, bm22)

    BLK_N = 1000
    hm = pl.pallas_call(
        _hm_body,
        grid=(N_NODES // BLK_N,),
        in_specs=[
            pl.BlockSpec((BLK_N, NODE_DIM), lambda i: (i, 0)),
            pl.BlockSpec((NODE_DIM, HIDDEN), lambda i: (0, 0)),
            pl.BlockSpec((1, HIDDEN), lambda i: (0, 0)),
            pl.BlockSpec((2 * HIDDEN, HIDDEN), lambda i: (0, 0)),
            pl.BlockSpec((1, HIDDEN), lambda i: (0, 0)),
        ],
        out_specs=pl.BlockSpec((BLK_N, HIDDEN), lambda i: (i, 0)),
        out_shape=jax.ShapeDtypeStruct((N_NODES, HIDDEN), F32),
    )(node_feats, Wn, bn2, Wm1, bias1)

    pad = E_PAD - N_EDGES
    src = jnp.concatenate(
        [edge_idx[0].astype(jnp.int32), jnp.zeros((pad,), jnp.int32)])
    dst = jnp.concatenate(
        [edge_idx[1].astype(jnp.int32), jnp.full((pad,), N_NODES, jnp.int32)])
    src3 = src.reshape(NW, K, CHUNK)
    dst3 = dst.reshape(NW, K, CHUNK)
    ef_p = jnp.concatenate(
        [edge_feats, jnp.zeros((pad, EDGE_DIM), F32)], axis=0)

    mesh = plsc.VectorSubcoreMesh(core_axis_name="c", subcore_axis_name="s")

    g = pl.kernel(
        _gather_body,
        out_type=jax.ShapeDtypeStruct((E_PAD, HIDDEN), F32),
        mesh=mesh,
        scratch_types=[
            pltpu.VMEM((K, CHUNK), jnp.int32),
            pltpu.VMEM((CHUNK, HIDDEN), F32),
            pltpu.VMEM((CHUNK, HIDDEN), F32),
            pltpu.VMEM((CHUNK, HIDDEN), F32),
            pltpu.SemaphoreType.DMA,
            pltpu.SemaphoreType.DMA,
            pltpu.SemaphoreType.DMA,
            pltpu.SemaphoreType.DMA,
            pltpu.SemaphoreType.DMA,
            pltpu.SemaphoreType.DMA,
            pltpu.VMEM_SHARED((N_NODES, HIDDEN), F32),
        ],
    )(hm, src3)

    BLK_E = 5632
    xt = pl.pallas_call(
        _edge_body,
        grid=(E_PAD // BLK_E,),
        in_specs=[
            pl.BlockSpec((BLK_E, HIDDEN), lambda i: (i, 0)),
            pl.BlockSpec((BLK_E, EDGE_DIM), lambda i: (i, 0)),
            pl.BlockSpec((EDGE_DIM, HIDDEN), lambda i: (0, 0)),
        ],
        out_specs=pl.BlockSpec((BLK_E, HIDDEN), lambda i: (i, 0)),
        out_shape=jax.ShapeDtypeStruct((E_PAD, HIDDEN), F32),
    )(g, ef_p, We2)

    zeros128 = jnp.zeros((N_PAD, HIDDEN), F32)
    agg2, degs = pl.kernel(
        _scatter_body,
        out_type=[
            jax.ShapeDtypeStruct((NC, N_PAD, HIDDEN), F32),
            jax.ShapeDtypeStruct((NW, N_PAD), F32),
        ],
        mesh=mesh,
        scratch_types=[
            pltpu.VMEM((K, CHUNK), jnp.int32),
            pltpu.VMEM((CHUNK, HIDDEN), F32),
            pltpu.VMEM((CHUNK, HIDDEN), F32),
            pltpu.VMEM((N_PAD,), F32),
            pltpu.SemaphoreType.DMA,
            pltpu.SemaphoreType.DMA,
            pltpu.VMEM_SHARED((N_PAD, HIDDEN), F32),
        ],
        compiler_params=pltpu.CompilerParams(needs_layout_passes=False),
    )(xt, dst3, zeros128)

    degterm = pl.pallas_call(
        _degterm_body,
        out_shape=jax.ShapeDtypeStruct((N_PAD, HIDDEN), F32),
    )(degs, bcombo)

    out = pl.pallas_call(
        _post_body,
        grid=(N_NODES // BLK_N,),
        in_specs=[
            pl.BlockSpec((BLK_N, NODE_DIM), lambda i: (i, 0)),
            pl.BlockSpec((NC, BLK_N, HIDDEN), lambda i: (0, i, 0)),
            pl.BlockSpec((BLK_N, HIDDEN), lambda i: (i, 0)),
            pl.BlockSpec((NODE_DIM + HIDDEN, HIDDEN), lambda i: (0, 0)),
            pl.BlockSpec((1, HIDDEN), lambda i: (0, 0)),
            pl.BlockSpec((HIDDEN, HIDDEN), lambda i: (0, 0)),
            pl.BlockSpec((HIDDEN, HIDDEN), lambda i: (0, 0)),
            pl.BlockSpec((1, HIDDEN), lambda i: (0, 0)),
        ],
        out_specs=pl.BlockSpec((BLK_N, HIDDEN), lambda i: (i, 0)),
        out_shape=jax.ShapeDtypeStruct((N_NODES, HIDDEN), F32),
    )(node_feats, agg2, degterm, Wu1, bu12, Wcombo, Wu2, bu22)

    return out


# speedup vs baseline: 1.1097x; 1.0775x over previous
import functools

import jax
import jax.numpy as jnp
from jax import lax
from jax.experimental import pallas as pl
from jax.experimental.pallas import tpu as pltpu
from jax.experimental.pallas import tpu_sc as plsc

N_NODES = 10000
N_EDGES = 320000
NODE_DIM = 128
EDGE_DIM = 16
HIDDEN = 128
N_PAD = N_NODES + 16

NC = 2
NS = 16
NW = NC * NS
CHUNK = 80
K = -(-N_EDGES // (NW * CHUNK))
E_PAD = NW * K * CHUNK

F32 = jnp.float32


def _prep_body(Wm1, We, be, bm1, Wm2, Wu1, bm2,
               We2_o, bias1_o, Wcombo_o, bcombo_o):
    Wm1_bot = Wm1[HIDDEN:, :]
    Wu1_bot = Wu1[NODE_DIM:, :]
    We2_o[...] = jnp.dot(We[...], Wm1_bot, preferred_element_type=F32)
    bias1_o[...] = bm1[...] + jnp.dot(be[...], Wm1_bot,
                                      preferred_element_type=F32)
    Wcombo_o[...] = jnp.dot(Wm2[...], Wu1_bot, preferred_element_type=F32)
    bcombo_o[...] = jnp.dot(bm2[...], Wu1_bot, preferred_element_type=F32)


def _hm_body(nf, Wn, bn, Wm1, bias1, hm_o):
    h = jnp.dot(nf[...], Wn[...], preferred_element_type=F32) + bn[...]
    hm_o[...] = jnp.dot(h, Wm1[:HIDDEN, :],
                        preferred_element_type=F32) + bias1[...]


def _edge_body(g, ef, We2, xt_o):
    xt_o[...] = jnp.maximum(
        g[...] + jnp.dot(ef[...], We2[...], preferred_element_type=F32), 0.0)


def _degterm_body(degs, bcombo, dt_o):
    bb = jnp.broadcast_to(bcombo[...], (NW, HIDDEN))
    dt_o[...] = lax.dot_general(degs[...], bb, (((0,), (0,)), ((), ())),
                                preferred_element_type=F32)


def _post_body(nf, agg2, degterm, Wu1, bu1, Wcombo, Wu2, bu2, out_o):
    aggs = agg2[0] + agg2[1]
    pre = (jnp.dot(nf[...], Wu1[:NODE_DIM, :], preferred_element_type=F32)
           + jnp.dot(aggs, Wcombo[...], preferred_element_type=F32)
           + degterm[...] + bu1[...])
    out_o[...] = jnp.dot(jnp.maximum(pre, 0.0), Wu2[...],
                         preferred_element_type=F32) + bu2[...]


def _gather_body(hm_hbm, idx_hbm, g_hbm, idx_v, buf0, buf1, buf2,
                 gsem0, gsem1, gsem2, wsem0, wsem1, wsem2, hm_sh):
    sid = lax.axis_index("s")
    wid = sid * NC + lax.axis_index("c")

    @pl.when(sid == 0)
    def _():
        pltpu.sync_copy(hm_hbm, hm_sh)

    plsc.subcore_barrier()
    pltpu.sync_copy(idx_hbm.at[wid], idx_v)
    base = wid * (K * CHUNK)
    bufs = (buf0, buf1, buf2)
    gsems = (gsem0, gsem1, gsem2)
    wsems = (wsem0, wsem1, wsem2)

    def out_slice(j):
        return g_hbm.at[pl.ds(base + j * CHUNK, CHUNK)]

    pltpu.async_copy(hm_sh.at[idx_v.at[0]], buf0, gsem0)

    def body(jj, carry):
        for t in range(3):
            j = jj * 3 + t
            b = t

            @pl.when(j < K)
            def _():
                pltpu.make_async_copy(
                    hm_sh.at[idx_v.at[j]], bufs[b], gsems[b]).wait()

                nb = (b + 1) % 3

                @pl.when(j + 1 < K)
                def _():
                    @pl.when(j >= 2)
                    def _():
                        pltpu.make_async_copy(
                            bufs[nb], out_slice(j - 2), wsems[nb]).wait()

                    pltpu.async_copy(
                        hm_sh.at[idx_v.at[j + 1]], bufs[nb], gsems[nb])

                pltpu.async_copy(bufs[b], out_slice(j), wsems[b])

        return carry

    lax.fori_loop(0, (K + 2) // 3, body, 0)
    for dd in (3, 2, 1):
        j = K - dd
        pltpu.make_async_copy(bufs[j % 3], out_slice(j), wsems[j % 3]).wait()


def _scatter_body(xt_hbm, idx_hbm, zero_hbm, agg_hbm, degs_hbm,
                  idx_v, buf0, buf1, deg_v, sem0, sem1, acc_sh):
    cid = lax.axis_index("c")
    sid = lax.axis_index("s")
    wid = sid * NC + cid

    @pl.when(sid == 0)
    def _():
        pltpu.sync_copy(zero_hbm, acc_sh)

    zvec = jnp.zeros((16,), F32)

    def zbody(i, carry):
        deg_v[pl.ds(i * 16, 16)] = zvec
        return carry

    lax.fori_loop(0, N_PAD // 16, zbody, 0)

    plsc.subcore_barrier()
    pltpu.sync_copy(idx_hbm.at[wid], idx_v)
    base = wid * (K * CHUNK)
    bufs = (buf0, buf1)
    sems = (sem0, sem1)
    ones = jnp.ones((16,), F32)

    pltpu.async_copy(xt_hbm.at[pl.ds(base, CHUNK)], buf0, sem0)

    def body(jj, carry):
        for t in range(2):
            j = jj * 2 + t
            b = t

            @pl.when(j < K)
            def _():
                pltpu.make_async_copy(
                    xt_hbm.at[pl.ds(base + j * CHUNK, CHUNK)],
                    bufs[b], sems[b]).wait()

                @pl.when(j + 1 < K)
                def _():
                    pltpu.async_copy(
                        xt_hbm.at[pl.ds(base + (j + 1) * CHUNK, CHUNK)],
                        bufs[1 - b], sems[1 - b])

                pltpu.sync_copy(bufs[b], acc_sh.at[idx_v.at[j]], add=True)

                for q in range(CHUNK // 16):
                    dvec = idx_v[j, pl.ds(q * 16, 16)]
                    plsc.addupdate_scatter(deg_v, [dvec], ones)

        return carry

    lax.fori_loop(0, (K + 1) // 2, body, 0)
    plsc.subcore_barrier()

    @pl.when(sid == 0)
    def _():
        pltpu.sync_copy(acc_sh, agg_hbm.at[cid])

    pltpu.sync_copy(deg_v, degs_hbm.at[wid])


def kernel(node_feats, edge_idx, edge_feats, Wn, bn, We, be,
           Wm1, bm1, Wm2, bm2, Wu1, bu1, Wu2, bu2):
    bn2 = bn.reshape(1, HIDDEN)
    be2 = be.reshape(1, HIDDEN)
    bm12 = bm1.reshape(1, HIDDEN)
    bm22 = bm2.reshape(1, HIDDEN)
    bu12 = bu1.reshape(1, HIDDEN)
    bu22 = bu2.reshape(1, HIDDEN)

    We2, bias1, Wcombo, bcombo = pl.pallas_call(
        _prep_body,
        out_shape=[
            jax.ShapeDtypeStruct((EDGE_DIM, HIDDEN), F32),
            jax.ShapeDtypeStruct((1, HIDDEN), F32),
            jax.ShapeDtypeStruct((HIDDEN, HIDDEN), F32),
            jax.ShapeDtypeStruct((1, HIDDEN), F32),
        ],
    )(Wm1, We, be2, bm12, Wm2, Wu1, bm22)

    BLK_N = 1000
    hm = pl.pallas_call(
        _hm_body,
        grid=(N_NODES // BLK_N,),
        in_specs=[
            pl.BlockSpec((BLK_N, NODE_DIM), lambda i: (i, 0)),
            pl.BlockSpec((NODE_DIM, HIDDEN), lambda i: (0, 0)),
            pl.BlockSpec((1, HIDDEN), lambda i: (0, 0)),
            pl.BlockSpec((2 * HIDDEN, HIDDEN), lambda i: (0, 0)),
            pl.BlockSpec((1, HIDDEN), lambda i: (0, 0)),
        ],
        out_specs=pl.BlockSpec((BLK_N, HIDDEN), lambda i: (i, 0)),
        out_shape=jax.ShapeDtypeStruct((N_NODES, HIDDEN), F32),
    )(node_feats, Wn, bn2, Wm1, bias1)

    src3 = edge_idx[0].astype(jnp.int32).reshape(NW, K, CHUNK)
    dst3 = edge_idx[1].astype(jnp.int32).reshape(NW, K, CHUNK)
    ef_p = edge_feats

    mesh = plsc.VectorSubcoreMesh(core_axis_name="c", subcore_axis_name="s")

    g = pl.kernel(
        _gather_body,
        out_type=jax.ShapeDtypeStruct((E_PAD, HIDDEN), F32),
        mesh=mesh,
        scratch_types=[
            pltpu.VMEM((K, CHUNK), jnp.int32),
            pltpu.VMEM((CHUNK, HIDDEN), F32),
            pltpu.VMEM((CHUNK, HIDDEN), F32),
            pltpu.VMEM((CHUNK, HIDDEN), F32),
            pltpu.SemaphoreType.DMA,
            pltpu.SemaphoreType.DMA,
            pltpu.SemaphoreType.DMA,
            pltpu.SemaphoreType.DMA,
            pltpu.SemaphoreType.DMA,
            pltpu.SemaphoreType.DMA,
            pltpu.VMEM_SHARED((N_NODES, HIDDEN), F32),
        ],
    )(hm, src3)

    BLK_E = 5000
    xt = pl.pallas_call(
        _edge_body,
        grid=(E_PAD // BLK_E,),
        in_specs=[
            pl.BlockSpec((BLK_E, HIDDEN), lambda i: (i, 0)),
            pl.BlockSpec((BLK_E, EDGE_DIM), lambda i: (i, 0)),
            pl.BlockSpec((EDGE_DIM, HIDDEN), lambda i: (0, 0)),
        ],
        out_specs=pl.BlockSpec((BLK_E, HIDDEN), lambda i: (i, 0)),
        out_shape=jax.ShapeDtypeStruct((E_PAD, HIDDEN), F32),
    )(g, ef_p, We2)

    zeros128 = jnp.zeros((N_PAD, HIDDEN), F32)
    agg2, degs = pl.kernel(
        _scatter_body,
        out_type=[
            jax.ShapeDtypeStruct((NC, N_PAD, HIDDEN), F32),
            jax.ShapeDtypeStruct((NW, N_PAD), F32),
        ],
        mesh=mesh,
        scratch_types=[
            pltpu.VMEM((K, CHUNK), jnp.int32),
            pltpu.VMEM((CHUNK, HIDDEN), F32),
            pltpu.VMEM((CHUNK, HIDDEN), F32),
            pltpu.VMEM((N_PAD,), F32),
            pltpu.SemaphoreType.DMA,
            pltpu.SemaphoreType.DMA,
            pltpu.VMEM_SHARED((N_PAD, HIDDEN), F32),
        ],
        compiler_params=pltpu.CompilerParams(needs_layout_passes=False),
    )(xt, dst3, zeros128)

    degterm = pl.pallas_call(
        _degterm_body,
        out_shape=jax.ShapeDtypeStruct((N_PAD, HIDDEN), F32),
    )(degs, bcombo)

    out = pl.pallas_call(
        _post_body,
        grid=(N_NODES // BLK_N,),
        in_specs=[
            pl.BlockSpec((BLK_N, NODE_DIM), lambda i: (i, 0)),
            pl.BlockSpec((NC, BLK_N, HIDDEN), lambda i: (0, i, 0)),
            pl.BlockSpec((BLK_N, HIDDEN), lambda i: (i, 0)),
            pl.BlockSpec((NODE_DIM + HIDDEN, HIDDEN), lambda i: (0, 0)),
            pl.BlockSpec((1, HIDDEN), lambda i: (0, 0)),
            pl.BlockSpec((HIDDEN, HIDDEN), lambda i: (0, 0)),
            pl.BlockSpec((HIDDEN, HIDDEN), lambda i: (0, 0)),
            pl.BlockSpec((1, HIDDEN), lambda i: (0, 0)),
        ],
        out_specs=pl.BlockSpec((BLK_N, HIDDEN), lambda i: (i, 0)),
        out_shape=jax.ShapeDtypeStruct((N_NODES, HIDDEN), F32),
    )(node_feats, agg2, degterm, Wu1, bu12, Wcombo, Wu2, bu22)

    return out


# speedup vs baseline: 1.1099x; 1.0002x over previous
import functools

import jax
import jax.numpy as jnp
from jax import lax
from jax.experimental import pallas as pl
from jax.experimental.pallas import tpu as pltpu
from jax.experimental.pallas import tpu_sc as plsc

N_NODES = 10000
N_EDGES = 320000
NODE_DIM = 128
EDGE_DIM = 16
HIDDEN = 128
N_PAD = N_NODES + 16

NC = 2
NS = 16
NW = NC * NS
CHUNK = 80
K = -(-N_EDGES // (NW * CHUNK))
E_PAD = NW * K * CHUNK

F32 = jnp.float32


def _prep_body(Wm1, We, be, bm1, Wm2, Wu1, bm2,
               We2_o, bias1_o, Wcombo_o, bcombo_o):
    Wm1_bot = Wm1[HIDDEN:, :]
    Wu1_bot = Wu1[NODE_DIM:, :]
    We2_o[...] = jnp.dot(We[...], Wm1_bot, preferred_element_type=F32)
    bias1_o[...] = bm1[...] + jnp.dot(be[...], Wm1_bot,
                                      preferred_element_type=F32)
    Wcombo_o[...] = jnp.dot(Wm2[...], Wu1_bot, preferred_element_type=F32)
    bcombo_o[...] = jnp.dot(bm2[...], Wu1_bot, preferred_element_type=F32)


def _hm_body(nf, Wn, bn, Wm1, bias1, hm_o):
    h = jnp.dot(nf[...], Wn[...], preferred_element_type=F32) + bn[...]
    hm_o[...] = jnp.dot(h, Wm1[:HIDDEN, :],
                        preferred_element_type=F32) + bias1[...]


def _edge_body(g, ef, We2, xt_o):
    xt_o[...] = jnp.maximum(
        g[...] + jnp.dot(ef[...], We2[...], preferred_element_type=F32), 0.0)


def _degterm_body(degs, bcombo, dt_o):
    bb = jnp.broadcast_to(bcombo[...], (NW, HIDDEN))
    dt_o[...] = lax.dot_general(degs[...], bb, (((0,), (0,)), ((), ())),
                                preferred_element_type=F32)


def _post_body(nf, agg2, degterm, Wu1, bu1, Wcombo, Wu2, bu2, out_o):
    aggs = agg2[0] + agg2[1]
    pre = (jnp.dot(nf[...], Wu1[:NODE_DIM, :], preferred_element_type=F32)
           + jnp.dot(aggs, Wcombo[...], preferred_element_type=F32)
           + degterm[...] + bu1[...])
    out_o[...] = jnp.dot(jnp.maximum(pre, 0.0), Wu2[...],
                         preferred_element_type=F32) + bu2[...]


def _gather_body(hm_hbm, idx_hbm, g_hbm, idx_v, buf0, buf1, buf2,
                 gsem0, gsem1, gsem2, wsem0, wsem1, wsem2, hm_sh):
    sid = lax.axis_index("s")
    wid = sid * NC + lax.axis_index("c")

    @pl.when(sid == 0)
    def _():
        pltpu.sync_copy(hm_hbm, hm_sh)

    plsc.subcore_barrier()
    pltpu.sync_copy(idx_hbm.at[wid], idx_v)
    base = wid * (K * CHUNK)
    bufs = (buf0, buf1, buf2)
    gsems = (gsem0, gsem1, gsem2)
    wsems = (wsem0, wsem1, wsem2)

    def out_slice(j):
        return g_hbm.at[pl.ds(base + j * CHUNK, CHUNK)]

    pltpu.async_copy(hm_sh.at[idx_v.at[0]], buf0, gsem0)

    def body(jj, carry):
        for t in range(3):
            j = jj * 3 + t
            b = t

            @pl.when(j < K)
            def _():
                pltpu.make_async_copy(
                    hm_sh.at[idx_v.at[j]], bufs[b], gsems[b]).wait()

                nb = (b + 1) % 3

                @pl.when(j + 1 < K)
                def _():
                    @pl.when(j >= 2)
                    def _():
                        pltpu.make_async_copy(
                            bufs[nb], out_slice(j - 2), wsems[nb]).wait()

                    pltpu.async_copy(
                        hm_sh.at[idx_v.at[j + 1]], bufs[nb], gsems[nb])

                pltpu.async_copy(bufs[b], out_slice(j), wsems[b])

        return carry

    lax.fori_loop(0, (K + 2) // 3, body, 0)
    for dd in (3, 2, 1):
        j = K - dd
        pltpu.make_async_copy(bufs[j % 3], out_slice(j), wsems[j % 3]).wait()


def _scatter_body(xt_hbm, idx_hbm, zero_hbm, agg_hbm, degs_hbm,
                  idx_v, buf0, buf1, deg_v, sem0, sem1, ssem0, ssem1,
                  acc_sh):
    cid = lax.axis_index("c")
    sid = lax.axis_index("s")
    wid = sid * NC + cid

    @pl.when(sid == 0)
    def _():
        pltpu.sync_copy(zero_hbm, acc_sh)

    zvec = jnp.zeros((16,), F32)

    def zbody(i, carry):
        deg_v[pl.ds(i * 16, 16)] = zvec
        return carry

    lax.fori_loop(0, N_PAD // 16, zbody, 0)

    plsc.subcore_barrier()
    pltpu.sync_copy(idx_hbm.at[wid], idx_v)
    base = wid * (K * CHUNK)
    bufs = (buf0, buf1)
    sems = (sem0, sem1)
    ssems = (ssem0, ssem1)
    ones = jnp.ones((16,), F32)

    pltpu.async_copy(xt_hbm.at[pl.ds(base, CHUNK)], buf0, sem0)

    def body(jj, carry):
        for t in range(2):
            j = jj * 2 + t
            b = t

            @pl.when(j < K)
            def _():
                pltpu.make_async_copy(
                    xt_hbm.at[pl.ds(base + j * CHUNK, CHUNK)],
                    bufs[b], sems[b]).wait()

                @pl.when(j + 1 < K)
                def _():
                    @pl.when(j >= 1)
                    def _():
                        pltpu.make_async_copy(
                            bufs[1 - b], acc_sh.at[idx_v.at[j - 1]],
                            ssems[1 - b]).wait()

                    pltpu.async_copy(
                        xt_hbm.at[pl.ds(base + (j + 1) * CHUNK, CHUNK)],
                        bufs[1 - b], sems[1 - b])

                pltpu.async_copy(bufs[b], acc_sh.at[idx_v.at[j]],
                                 ssems[b], add=True)

                for q in range(CHUNK // 16):
                    dvec = idx_v[j, pl.ds(q * 16, 16)]
                    plsc.addupdate_scatter(deg_v, [dvec], ones)

        return carry

    lax.fori_loop(0, (K + 1) // 2, body, 0)
    pltpu.make_async_copy(bufs[(K - 2) % 2], acc_sh.at[idx_v.at[K - 2]],
                          ssems[(K - 2) % 2]).wait()
    pltpu.make_async_copy(bufs[(K - 1) % 2], acc_sh.at[idx_v.at[K - 1]],
                          ssems[(K - 1) % 2]).wait()
    plsc.subcore_barrier()

    @pl.when(sid == 0)
    def _():
        pltpu.sync_copy(acc_sh, agg_hbm.at[cid])

    pltpu.sync_copy(deg_v, degs_hbm.at[wid])


def kernel(node_feats, edge_idx, edge_feats, Wn, bn, We, be,
           Wm1, bm1, Wm2, bm2, Wu1, bu1, Wu2, bu2):
    bn2 = bn.reshape(1, HIDDEN)
    be2 = be.reshape(1, HIDDEN)
    bm12 = bm1.reshape(1, HIDDEN)
    bm22 = bm2.reshape(1, HIDDEN)
    bu12 = bu1.reshape(1, HIDDEN)
    bu22 = bu2.reshape(1, HIDDEN)

    We2, bias1, Wcombo, bcombo = pl.pallas_call(
        _prep_body,
        out_shape=[
            jax.ShapeDtypeStruct((EDGE_DIM, HIDDEN), F32),
            jax.ShapeDtypeStruct((1, HIDDEN), F32),
            jax.ShapeDtypeStruct((HIDDEN, HIDDEN), F32),
            jax.ShapeDtypeStruct((1, HIDDEN), F32),
        ],
    )(Wm1, We, be2, bm12, Wm2, Wu1, bm22)

    BLK_N = 1000
    hm = pl.pallas_call(
        _hm_body,
        grid=(N_NODES // BLK_N,),
        in_specs=[
            pl.BlockSpec((BLK_N, NODE_DIM), lambda i: (i, 0)),
            pl.BlockSpec((NODE_DIM, HIDDEN), lambda i: (0, 0)),
            pl.BlockSpec((1, HIDDEN), lambda i: (0, 0)),
            pl.BlockSpec((2 * HIDDEN, HIDDEN), lambda i: (0, 0)),
            pl.BlockSpec((1, HIDDEN), lambda i: (0, 0)),
        ],
        out_specs=pl.BlockSpec((BLK_N, HIDDEN), lambda i: (i, 0)),
        out_shape=jax.ShapeDtypeStruct((N_NODES, HIDDEN), F32),
    )(node_feats, Wn, bn2, Wm1, bias1)

    src3 = edge_idx[0].astype(jnp.int32).reshape(NW, K, CHUNK)
    dst3 = edge_idx[1].astype(jnp.int32).reshape(NW, K, CHUNK)
    ef_p = edge_feats

    mesh = plsc.VectorSubcoreMesh(core_axis_name="c", subcore_axis_name="s")

    g = pl.kernel(
        _gather_body,
        out_type=jax.ShapeDtypeStruct((E_PAD, HIDDEN), F32),
        mesh=mesh,
        scratch_types=[
            pltpu.VMEM((K, CHUNK), jnp.int32),
            pltpu.VMEM((CHUNK, HIDDEN), F32),
            pltpu.VMEM((CHUNK, HIDDEN), F32),
            pltpu.VMEM((CHUNK, HIDDEN), F32),
            pltpu.SemaphoreType.DMA,
            pltpu.SemaphoreType.DMA,
            pltpu.SemaphoreType.DMA,
            pltpu.SemaphoreType.DMA,
            pltpu.SemaphoreType.DMA,
            pltpu.SemaphoreType.DMA,
            pltpu.VMEM_SHARED((N_NODES, HIDDEN), F32),
        ],
    )(hm, src3)

    BLK_E = 5000
    xt = pl.pallas_call(
        _edge_body,
        grid=(E_PAD // BLK_E,),
        in_specs=[
            pl.BlockSpec((BLK_E, HIDDEN), lambda i: (i, 0)),
            pl.BlockSpec((BLK_E, EDGE_DIM), lambda i: (i, 0)),
            pl.BlockSpec((EDGE_DIM, HIDDEN), lambda i: (0, 0)),
        ],
        out_specs=pl.BlockSpec((BLK_E, HIDDEN), lambda i: (i, 0)),
        out_shape=jax.ShapeDtypeStruct((E_PAD, HIDDEN), F32),
    )(g, ef_p, We2)

    zeros128 = jnp.zeros((N_PAD, HIDDEN), F32)
    agg2, degs = pl.kernel(
        _scatter_body,
        out_type=[
            jax.ShapeDtypeStruct((NC, N_PAD, HIDDEN), F32),
            jax.ShapeDtypeStruct((NW, N_PAD), F32),
        ],
        mesh=mesh,
        scratch_types=[
            pltpu.VMEM((K, CHUNK), jnp.int32),
            pltpu.VMEM((CHUNK, HIDDEN), F32),
            pltpu.VMEM((CHUNK, HIDDEN), F32),
            pltpu.VMEM((N_PAD,), F32),
            pltpu.SemaphoreType.DMA,
            pltpu.SemaphoreType.DMA,
            pltpu.SemaphoreType.DMA,
            pltpu.SemaphoreType.DMA,
            pltpu.VMEM_SHARED((N_PAD, HIDDEN), F32),
        ],
        compiler_params=pltpu.CompilerParams(needs_layout_passes=False),
    )(xt, dst3, zeros128)

    degterm = pl.pallas_call(
        _degterm_body,
        out_shape=jax.ShapeDtypeStruct((N_PAD, HIDDEN), F32),
    )(degs, bcombo)

    out = pl.pallas_call(
        _post_body,
        grid=(N_NODES // BLK_N,),
        in_specs=[
            pl.BlockSpec((BLK_N, NODE_DIM), lambda i: (i, 0)),
            pl.BlockSpec((NC, BLK_N, HIDDEN), lambda i: (0, i, 0)),
            pl.BlockSpec((BLK_N, HIDDEN), lambda i: (i, 0)),
            pl.BlockSpec((NODE_DIM + HIDDEN, HIDDEN), lambda i: (0, 0)),
            pl.BlockSpec((1, HIDDEN), lambda i: (0, 0)),
            pl.BlockSpec((HIDDEN, HIDDEN), lambda i: (0, 0)),
            pl.BlockSpec((HIDDEN, HIDDEN), lambda i: (0, 0)),
            pl.BlockSpec((1, HIDDEN), lambda i: (0, 0)),
        ],
        out_specs=pl.BlockSpec((BLK_N, HIDDEN), lambda i: (i, 0)),
        out_shape=jax.ShapeDtypeStruct((N_NODES, HIDDEN), F32),
    )(node_feats, agg2, degterm, Wu1, bu12, Wcombo, Wu2, bu22)

    return out
